# Initial kernel scaffold; baseline (speedup 1.0000x reference)
import jax
import jax.numpy as jnp
from jax.experimental import pallas as pl


def kernel(patch):
    raise NotImplementedError("write your pallas kernel here")


# speedup vs baseline: 2.3685x; 2.3685x over previous
import functools

import jax
import jax.numpy as jnp
import numpy as np
from jax import lax
from jax.experimental import pallas as pl
from jax.experimental.pallas import tpu as pltpu
from jax.experimental.pallas import tpu_sc as plsc

SIZE = 384
PATCH = 16
NUM = SIZE // PATCH
C = 32
D = 768
P = NUM * NUM
NAUG = 8

_info = plsc.get_sparse_core_info()
NC, NS = _info.num_cores, _info.num_subcores
NW = NC * NS

TOTAL_ROWS = NAUG * C * P
ROWS_PER_W = TOTAL_ROWS // NW
K = 64
NCHUNK = ROWS_PER_W // K


def _build_tables():
    grid = np.arange(P, dtype=np.int32).reshape(NUM, NUM)
    idx_list, srt_list = [], []
    for k in range(4):
        rg = np.rot90(grid, k=k, axes=(0, 1))
        flat = rg.flatten()
        idx_list.append(flat)
        srt_list.append(np.argsort(flat))
        fl = np.flip(rg, axis=1).flatten()
        idx_list.append(fl)
        srt_list.append(np.argsort(fl))
    idx = np.stack(idx_list).astype(np.int32)
    srt = np.stack(srt_list).astype(np.int32)
    rows = idx[:, None, :] + (np.arange(C, dtype=np.int32) * P)[None, :, None]
    rows = rows.reshape(NW, NCHUNK, K)
    return rows, srt


_SRC_ROWS_NP, _ARGSORT_NP = _build_tables()


@functools.partial(
    pl.kernel,
    mesh=plsc.VectorSubcoreMesh(core_axis_name="c", subcore_axis_name="s"),
    out_type=jax.ShapeDtypeStruct((TOTAL_ROWS, D), jnp.float32),
    scratch_types=[
        pltpu.VMEM((NCHUNK, K), jnp.int32),
        pltpu.VMEM((K, D), jnp.float32),
        pltpu.SemaphoreType.DMA,
    ],
)
def _gather_rows(pf_hbm, idx_hbm, out_hbm, idx_v, buf, sem):
    wid = lax.axis_index("s") * NC + lax.axis_index("c")
    pltpu.sync_copy(idx_hbm.at[wid], idx_v)

    def body(t, carry):
        pltpu.async_copy(pf_hbm.at[idx_v.at[t]], buf, sem).wait()
        base = pl.multiple_of(wid * ROWS_PER_W + t * K, K)
        pltpu.sync_copy(buf, out_hbm.at[pl.ds(base, K)])
        return carry

    lax.fori_loop(0, NCHUNK, body, 0)


def kernel(patch):
    pf = patch.reshape(C * P, D)
    idx = jnp.asarray(_SRC_ROWS_NP)
    out_flat = _gather_rows(pf, idx)
    aug = out_flat.reshape(NAUG, C, P, D)
    argsort = jnp.asarray(_ARGSORT_NP)
    perm = jnp.arange(NAUG, dtype=jnp.int32)
    return aug, argsort, perm


# speedup vs baseline: 2.6896x; 1.1356x over previous
import functools

import jax
import jax.numpy as jnp
import numpy as np
from jax import lax
from jax.experimental import pallas as pl
from jax.experimental.pallas import tpu as pltpu
from jax.experimental.pallas import tpu_sc as plsc

SIZE = 384
PATCH = 16
NUM = SIZE // PATCH
C = 32
D = 768
P = NUM * NUM
NAUG = 8

_info = plsc.get_sparse_core_info()
NC, NS = _info.num_cores, _info.num_subcores
NW = NC * NS

TOTAL_ROWS = NAUG * C * P
ROWS_PER_W = TOTAL_ROWS // NW
K = 64
NCHUNK = ROWS_PER_W // K


def _build_tables():
    grid = np.arange(P, dtype=np.int32).reshape(NUM, NUM)
    idx_list, srt_list = [], []
    for k in range(4):
        rg = np.rot90(grid, k=k, axes=(0, 1))
        flat = rg.flatten()
        idx_list.append(flat)
        srt_list.append(np.argsort(flat))
        fl = np.flip(rg, axis=1).flatten()
        idx_list.append(fl)
        srt_list.append(np.argsort(fl))
    idx = np.stack(idx_list).astype(np.int32)
    srt = np.stack(srt_list).astype(np.int32)
    rows = idx[:, None, :] + (np.arange(C, dtype=np.int32) * P)[None, :, None]
    rows = rows.reshape(NW, NCHUNK, K)
    return rows, srt


_SRC_ROWS_NP, _ARGSORT_NP = _build_tables()


@functools.partial(
    pl.kernel,
    mesh=plsc.VectorSubcoreMesh(core_axis_name="c", subcore_axis_name="s"),
    out_type=jax.ShapeDtypeStruct((TOTAL_ROWS, D), jnp.float32),
    scratch_types=[
        pltpu.VMEM((NCHUNK, K), jnp.int32),
        pltpu.VMEM((K, D), jnp.float32),
        pltpu.VMEM((K, D), jnp.float32),
        pltpu.SemaphoreType.DMA,
        pltpu.SemaphoreType.DMA,
    ],
)
def _gather_rows(pf_hbm, idx_hbm, out_hbm, idx_v, buf0, buf1, sem0, sem1):
    wid = lax.axis_index("s") * NC + lax.axis_index("c")
    pltpu.sync_copy(idx_hbm.at[wid], idx_v)
    row0 = wid * ROWS_PER_W

    pltpu.async_copy(pf_hbm.at[idx_v.at[0]], buf0, sem0)

    def body(g, carry):
        t0 = 2 * g
        t1 = t0 + 1
        pltpu.async_copy(pf_hbm.at[idx_v.at[t1]], buf1, sem1)
        pltpu.make_async_copy(pf_hbm.at[idx_v.at[t0]], buf0, sem0).wait()
        base0 = pl.multiple_of(row0 + t0 * K, K)
        pltpu.sync_copy(buf0, out_hbm.at[pl.ds(base0, K)])

        @pl.when(t0 + 2 < NCHUNK)
        def _():
            pltpu.async_copy(pf_hbm.at[idx_v.at[t0 + 2]], buf0, sem0)

        pltpu.make_async_copy(pf_hbm.at[idx_v.at[t1]], buf1, sem1).wait()
        base1 = pl.multiple_of(row0 + t1 * K, K)
        pltpu.sync_copy(buf1, out_hbm.at[pl.ds(base1, K)])
        return carry

    lax.fori_loop(0, NCHUNK // 2, body, 0)


def kernel(patch):
    pf = patch.reshape(C * P, D)
    idx = jnp.asarray(_SRC_ROWS_NP)
    out_flat = _gather_rows(pf, idx)
    aug = out_flat.reshape(NAUG, C, P, D)
    argsort = jnp.asarray(_ARGSORT_NP)
    perm = jnp.arange(NAUG, dtype=jnp.int32)
    return aug, argsort, perm


# speedup vs baseline: 4.6984x; 1.7469x over previous
import functools

import jax
import jax.numpy as jnp
import numpy as np
from jax import lax
from jax.experimental import pallas as pl
from jax.experimental.pallas import tpu as pltpu
from jax.experimental.pallas import tpu_sc as plsc

SIZE = 384
PATCH = 16
NUM = SIZE // PATCH
C = 32
D = 768
P = NUM * NUM
NAUG = 8

_info = plsc.get_sparse_core_info()
NC, NS = _info.num_cores, _info.num_subcores
NW = NC * NS

TOTAL_ROWS = NAUG * C * P
K = 72
NCHUNK = P // K


def _build_tables():
    grid = np.arange(P, dtype=np.int32).reshape(NUM, NUM)
    srt_list = []
    for k in range(4):
        rg = np.rot90(grid, k=k, axes=(0, 1))
        srt_list.append(np.argsort(rg.flatten()))
        srt_list.append(np.argsort(np.flip(rg, axis=1).flatten()))
    srt = np.stack(srt_list).astype(np.int32)
    a_base = (np.arange(NAUG, dtype=np.int32) * (C * P))[None, None, :, None]
    c_base = (np.arange(C, dtype=np.int32) * P)[:, None, None, None]
    pos = srt.reshape(NAUG, NCHUNK, K).transpose(1, 0, 2)[None]
    out_idx = a_base + c_base + pos
    return out_idx.reshape(C, NCHUNK * NAUG, K).astype(np.int32), srt


_OUT_IDX_NP, _ARGSORT_NP = _build_tables()


@functools.partial(
    pl.kernel,
    mesh=plsc.VectorSubcoreMesh(core_axis_name="c", subcore_axis_name="s"),
    out_type=jax.ShapeDtypeStruct((TOTAL_ROWS, D), jnp.float32),
    scratch_types=[
        pltpu.VMEM((NCHUNK * NAUG, K), jnp.int32),
        pltpu.VMEM((K, D), jnp.float32),
        pltpu.VMEM((K, D), jnp.float32),
        pltpu.SemaphoreType.DMA,
        pltpu.SemaphoreType.DMA,
        pltpu.SemaphoreType.DMA,
        pltpu.SemaphoreType.DMA,
    ],
)
def _scatter_augs(pf_hbm, idx_hbm, out_hbm, idx_v, buf0, buf1, rs0, rs1,
                  ss0, ss1):
    wid = lax.axis_index("s") * NC + lax.axis_index("c")
    pltpu.sync_copy(idx_hbm.at[wid], idx_v)
    src0 = pl.multiple_of(wid * P, P)

    def read(t, buf, sem):
        pltpu.async_copy(pf_hbm.at[pl.ds(src0 + t * K, K)], buf, sem)

    def scatter_all(t, buf, sem):
        for a in range(NAUG):
            pltpu.async_copy(buf, out_hbm.at[idx_v.at[t * NAUG + a]], sem)

    def drain_all(t, buf, sem):
        for a in range(NAUG):
            pltpu.make_async_copy(buf, out_hbm.at[idx_v.at[t * NAUG + a]],
                                  sem).wait()

    read(0, buf0, rs0)

    def body(g, carry):
        t0 = 2 * g
        t1 = t0 + 1
        read(t1, buf1, rs1)
        pltpu.make_async_copy(pf_hbm.at[pl.ds(src0 + t0 * K, K)], buf0,
                              rs0).wait()
        scatter_all(t0, buf0, ss0)
        drain_all(t0, buf0, ss0)

        @pl.when(t0 + 2 < NCHUNK)
        def _():
            read(t0 + 2, buf0, rs0)

        pltpu.make_async_copy(pf_hbm.at[pl.ds(src0 + t1 * K, K)], buf1,
                              rs1).wait()
        scatter_all(t1, buf1, ss1)
        drain_all(t1, buf1, ss1)
        return carry

    lax.fori_loop(0, NCHUNK // 2, body, 0)


def kernel(patch):
    pf = patch.reshape(C * P, D)
    idx = jnp.asarray(_OUT_IDX_NP)
    out_flat = _scatter_augs(pf, idx)
    aug = out_flat.reshape(NAUG, C, P, D)
    argsort = jnp.asarray(_ARGSORT_NP)
    perm = jnp.arange(NAUG, dtype=jnp.int32)
    return aug, argsort, perm


# speedup vs baseline: 4.7629x; 1.0137x over previous
import functools

import jax
import jax.numpy as jnp
import numpy as np
from jax import lax
from jax.experimental import pallas as pl
from jax.experimental.pallas import tpu as pltpu
from jax.experimental.pallas import tpu_sc as plsc

SIZE = 384
PATCH = 16
NUM = SIZE // PATCH
C = 32
D = 768
P = NUM * NUM
NAUG = 8

_info = plsc.get_sparse_core_info()
NC, NS = _info.num_cores, _info.num_subcores
NW = NC * NS

TOTAL_ROWS = NAUG * C * P
K = 72
NCHUNK = P // K


def _build_tables():
    grid = np.arange(P, dtype=np.int32).reshape(NUM, NUM)
    srt_list = []
    for k in range(4):
        rg = np.rot90(grid, k=k, axes=(0, 1))
        srt_list.append(np.argsort(rg.flatten()))
        srt_list.append(np.argsort(np.flip(rg, axis=1).flatten()))
    srt = np.stack(srt_list).astype(np.int32)
    a_base = (np.arange(NAUG, dtype=np.int32) * (C * P))[None, None, :, None]
    c_base = (np.arange(C, dtype=np.int32) * P)[:, None, None, None]
    pos = srt.reshape(NAUG, NCHUNK, K).transpose(1, 0, 2)[None]
    out_idx = a_base + c_base + pos
    return out_idx.reshape(C, NCHUNK * NAUG, K).astype(np.int32), srt


_OUT_IDX_NP, _ARGSORT_NP = _build_tables()


@functools.partial(
    pl.kernel,
    mesh=plsc.VectorSubcoreMesh(core_axis_name="c", subcore_axis_name="s"),
    out_type=jax.ShapeDtypeStruct((TOTAL_ROWS, D), jnp.float32),
    scratch_types=[
        pltpu.VMEM((NCHUNK * NAUG, K), jnp.int32),
        pltpu.VMEM((K, D), jnp.float32),
        pltpu.VMEM((K, D), jnp.float32),
        pltpu.SemaphoreType.DMA,
        pltpu.SemaphoreType.DMA,
        pltpu.SemaphoreType.DMA,
        pltpu.SemaphoreType.DMA,
    ],
)
def _scatter_augs(pf_hbm, idx_hbm, out_hbm, idx_v, buf0, buf1, rs0, rs1,
                  ss0, ss1):
    wid = lax.axis_index("s") * NC + lax.axis_index("c")
    pltpu.sync_copy(idx_hbm.at[wid], idx_v)
    src0 = pl.multiple_of(wid * P, P)

    def read(t, buf, sem):
        pltpu.async_copy(pf_hbm.at[pl.ds(src0 + t * K, K)], buf, sem)

    def scatter_all(t, buf, sem):
        for a in range(NAUG):
            pltpu.async_copy(buf, out_hbm.at[idx_v.at[t * NAUG + a]], sem)

    def drain_all(t, buf, sem):
        for a in range(NAUG):
            pltpu.make_async_copy(buf, out_hbm.at[idx_v.at[t * NAUG + a]],
                                  sem).wait()

    read(0, buf0, rs0)
    read(1, buf1, rs1)

    def body(g, carry):
        t0 = 2 * g
        t1 = t0 + 1
        pltpu.make_async_copy(pf_hbm.at[pl.ds(src0 + t0 * K, K)], buf0,
                              rs0).wait()
        scatter_all(t0, buf0, ss0)
        pltpu.make_async_copy(pf_hbm.at[pl.ds(src0 + t1 * K, K)], buf1,
                              rs1).wait()
        scatter_all(t1, buf1, ss1)
        drain_all(t0, buf0, ss0)

        @pl.when(t0 + 2 < NCHUNK)
        def _():
            read(t0 + 2, buf0, rs0)

        drain_all(t1, buf1, ss1)

        @pl.when(t1 + 2 < NCHUNK)
        def _():
            read(t1 + 2, buf1, rs1)

        return carry

    lax.fori_loop(0, NCHUNK // 2, body, 0)


def kernel(patch):
    pf = patch.reshape(C * P, D)
    idx = jnp.asarray(_OUT_IDX_NP)
    out_flat = _scatter_augs(pf, idx)
    aug = out_flat.reshape(NAUG, C, P, D)
    argsort = jnp.asarray(_ARGSORT_NP)
    perm = jnp.arange(NAUG, dtype=jnp.int32)
    return aug, argsort, perm
